# baseline (speedup 1.0000x reference)
import functools

import jax
import jax.numpy as jnp
from jax import lax
from jax.experimental import pallas as pl
from jax.experimental.pallas import tpu as pltpu
from jax.experimental.pallas import tpu_sc as plsc

N, E, D, R, B = 10000, 160000, 128, 64, 8
N_PAD = 10240
EH = E // 2
EH_P = 81920
TE = 2560
TN = 640
CHUNK = 128
NC, NS = 2, 16
NW = NC * NS
NJ = EH_P // CHUNK // NW
NSLOT = 4
LOOK = 2
ZROWS = N_PAD // NS


SSLOT = 2
_IDX_SCRATCH = [pltpu.VMEM((CHUNK,), jnp.int32) for _ in range(NSLOT)]
_IDX_SCRATCH2 = [pltpu.VMEM((CHUNK,), jnp.int32) for _ in range(SSLOT)]


def _sc_gather(x_pad, src):
    mesh = plsc.VectorSubcoreMesh(core_axis_name="c", subcore_axis_name="s")

    @functools.partial(
        pl.kernel, mesh=mesh,
        out_type=jax.ShapeDtypeStruct((EH_P, D), jnp.float32),
        scratch_types=[
            *_IDX_SCRATCH,
            pltpu.VMEM((NSLOT, CHUNK, D), jnp.float32),
            pltpu.SemaphoreType.DMA((NSLOT,)),
            pltpu.SemaphoreType.DMA((NSLOT,)),
            pltpu.SemaphoreType.DMA((NSLOT,)),
        ],
    )
    def k(x_hbm, src_hbm, out_hbm, i0, i1, i2, i3, rows, sem_i, sem_g, sem_o):
        idx = [i0, i1, i2, i3]
        c = lax.axis_index("c")
        s = lax.axis_index("s")
        wid = s * NC + c
        start = wid * NJ
        ih = [None] * NJ
        oh = [None] * NJ
        for t in range(NJ + 1):
            if t < NJ:
                b = t % NSLOT
                if t >= NSLOT:
                    oh[t - NSLOT].wait()
                ih[t] = pltpu.async_copy(
                    src_hbm.at[pl.ds((start + t) * CHUNK, CHUNK)],
                    idx[b], sem_i.at[b])
            tg = t - 1
            if 0 <= tg < NJ:
                bg = tg % NSLOT
                ih[tg].wait()
                pltpu.async_copy(
                    x_hbm.at[idx[bg]], rows.at[bg], sem_g.at[bg]).wait()
                oh[tg] = pltpu.async_copy(
                    rows.at[bg],
                    out_hbm.at[pl.ds((start + tg) * CHUNK, CHUNK)],
                    sem_o.at[bg])
        for to in range(NJ - NSLOT, NJ):
            oh[to].wait()

    return k(x_pad, src)


def _sc_scatter(msg, dst, zrows):
    mesh = plsc.VectorSubcoreMesh(core_axis_name="c", subcore_axis_name="s")

    @functools.partial(
        pl.kernel, mesh=mesh,
        out_type=jax.ShapeDtypeStruct((NC, N_PAD, D), jnp.float32),
        scratch_types=[
            pltpu.VMEM_SHARED((N_PAD, D), jnp.float32),
            *_IDX_SCRATCH2,
            pltpu.VMEM((SSLOT, CHUNK, D), jnp.float32),
            pltpu.SemaphoreType.DMA((SSLOT,)),
            pltpu.SemaphoreType.DMA((SSLOT,)),
            pltpu.SemaphoreType.DMA((SSLOT,)),
        ],
    )
    def k(msg_hbm, dst_hbm, zero_hbm, out_hbm, accum, i0, i1, rows,
          sem_i, sem_m, sem_a):
        idx = [i0, i1]
        c = lax.axis_index("c")
        s = lax.axis_index("s")
        wid = s * NC + c
        start = wid * NJ
        pltpu.sync_copy(zero_hbm, accum.at[pl.ds(s * ZROWS, ZROWS)])
        plsc.subcore_barrier()
        ih = [None] * NJ
        mh = [None] * NJ
        ah = [None] * NJ
        for t in range(NJ + 1):
            if t < NJ:
                b = t % SSLOT
                if t >= SSLOT:
                    ah[t - SSLOT].wait()
                ih[t] = pltpu.async_copy(
                    dst_hbm.at[pl.ds((start + t) * CHUNK, CHUNK)],
                    idx[b], sem_i.at[b])
                mh[t] = pltpu.async_copy(
                    msg_hbm.at[pl.ds((start + t) * CHUNK, CHUNK)],
                    rows.at[b], sem_m.at[b])
            ta = t - 1
            if 0 <= ta < NJ:
                ba = ta % SSLOT
                ih[ta].wait()
                mh[ta].wait()
                ah[ta] = pltpu.async_copy(
                    rows.at[ba], accum.at[idx[ba]], sem_a.at[ba], add=True)
        for ta in range(NJ - SSLOT, NJ):
            ah[ta].wait()
        plsc.subcore_barrier()
        pltpu.sync_copy(accum.at[pl.ds(s * ZROWS, ZROWS)],
                        out_hbm.at[c, pl.ds(s * ZROWS, ZROWS)])

    return k(msg, dst, zrows)


def _msg_body(xs_ref, r_ref, norm_ref, basis_ref, comp_ref, out_ref):
    xt = xs_ref[:].astype(jnp.bfloat16)
    rt = r_ref[0]
    nt = norm_ref[0]
    onehot_t = (rt == lax.broadcasted_iota(jnp.int32, (R, TE), 0))
    onehot_t = (onehot_t.astype(jnp.float32) * nt).astype(jnp.bfloat16)
    cw = lax.dot_general(onehot_t, comp_ref[:],
                         (((0,), (0,)), ((), ())),
                         preferred_element_type=jnp.float32).astype(jnp.bfloat16)
    xnw = jnp.concatenate([xt] * B, axis=1) * cw
    out_ref[:] = jnp.dot(xnw, basis_ref[:],
                         preferred_element_type=jnp.float32)


def _tc_msg(xs, r3, norm3, basis_flat, comp_wide):
    return pl.pallas_call(
        _msg_body,
        grid=(EH_P // TE,),
        in_specs=[
            pl.BlockSpec((TE, D), lambda i: (i, 0)),
            pl.BlockSpec((1, 1, TE), lambda i: (i, 0, 0)),
            pl.BlockSpec((1, 1, TE), lambda i: (i, 0, 0)),
            pl.BlockSpec((B * D, D), lambda i: (0, 0)),
            pl.BlockSpec((R, B * D), lambda i: (0, 0)),
        ],
        out_specs=pl.BlockSpec((TE, D), lambda i: (i, 0)),
        out_shape=jax.ShapeDtypeStruct((EH_P, D), jnp.float32),
    )(xs, r3, norm3, basis_flat, comp_wide)


def _final_body(act, pa_ref, pb_ref, x_ref, loop_ref, bias_ref, out_ref):
    pre = (pa_ref[0] + pa_ref[1] + pb_ref[0] + pb_ref[1]
           + jnp.dot(x_ref[:], loop_ref[:], preferred_element_type=jnp.float32)
           + bias_ref[:])
    out_ref[:] = act(pre)


def _tc_final(parts_a, parts_b, x_pad, loop_w, bias2d, act):
    return pl.pallas_call(
        functools.partial(_final_body, act),
        grid=(N_PAD // TN,),
        in_specs=[
            pl.BlockSpec((NC, TN, D), lambda i: (0, i, 0)),
            pl.BlockSpec((NC, TN, D), lambda i: (0, i, 0)),
            pl.BlockSpec((TN, D), lambda i: (i, 0)),
            pl.BlockSpec((D, D), lambda i: (0, 0)),
            pl.BlockSpec((1, D), lambda i: (0, 0)),
        ],
        out_specs=pl.BlockSpec((TN, D), lambda i: (i, 0)),
        out_shape=jax.ShapeDtypeStruct((N_PAD, D), jnp.float32),
    )(parts_a, parts_b, x_pad, loop_w, bias2d)


def _layer(x_pad, halves, zrows, basis, comp, loop_w, bias, act):
    basis_flat = basis.reshape(B * D, D).astype(jnp.bfloat16)
    comp_wide = jnp.repeat(comp, D, axis=1).astype(jnp.bfloat16)
    msgs = [None, None]
    for i, (src_h, dst_h, r3_h, n3_h) in enumerate(halves):
        xs = _sc_gather(x_pad, src_h)
        msgs[i] = _tc_msg(xs, r3_h, n3_h, basis_flat, comp_wide)
    parts = [_sc_scatter(msgs[i], h[1], zrows) for i, h in enumerate(halves)]
    return _tc_final(parts[0], parts[1], x_pad, loop_w,
                     bias.reshape(1, D), act)


def kernel(h, edge_index, r, norm, emb, basis1, comp1, loop1, bias1,
           basis2, comp2, loop2, bias2):
    src = edge_index[0]
    dst = edge_index[1]
    x = jnp.take(emb, h, axis=0)
    x_pad = jnp.pad(x, ((0, N_PAD - N), (0, 0)))
    pad_e = EH_P - EH
    halves = []
    for i in range(2):
        sl = slice(i * EH, (i + 1) * EH)
        src_p = jnp.pad(src[sl], (0, pad_e))
        dst_p = jnp.pad(dst[sl], (0, pad_e))
        r_p = jnp.pad(r[sl], (0, pad_e)).reshape(EH_P // TE, 1, TE)
        n_p = jnp.pad(norm[sl, 0], (0, pad_e)).reshape(EH_P // TE, 1, TE)
        halves.append((src_p, dst_p, r_p, n_p))
    zrows = jnp.zeros((ZROWS, D), jnp.float32)
    x_pad = _layer(x_pad, halves, zrows, basis1, comp1, loop1, bias1,
                   jax.nn.relu)
    x_pad = _layer(x_pad, halves, zrows, basis2, comp2, loop2, bias2,
                   jax.nn.sigmoid)
    return x_pad[:N]


# speedup vs baseline: 1.0810x; 1.0810x over previous
import functools

import jax
import jax.numpy as jnp
from jax import lax
from jax.experimental import pallas as pl
from jax.experimental.pallas import tpu as pltpu
from jax.experimental.pallas import tpu_sc as plsc

N, E, D, R, B = 10000, 160000, 128, 64, 8
N_PAD = 10240
EH = E // 2
EH_P = 81920
TE = 2560
TN = 640
CHUNK = 128
NC, NS = 2, 16
NW = NC * NS
NJ = EH_P // CHUNK // NW
NSLOT = 4
LOOK = 2
ZROWS = N_PAD // NS


SSLOT = 2
_IDX_SCRATCH = [pltpu.VMEM((CHUNK,), jnp.int32) for _ in range(NSLOT)]
_IDX_SCRATCH2 = [pltpu.VMEM((CHUNK,), jnp.int32) for _ in range(SSLOT)]


def _sc_gather(x_pad, src):
    mesh = plsc.VectorSubcoreMesh(core_axis_name="c", subcore_axis_name="s")

    @functools.partial(
        pl.kernel, mesh=mesh,
        out_type=jax.ShapeDtypeStruct((EH_P, D), jnp.float32),
        scratch_types=[
            *_IDX_SCRATCH,
            pltpu.VMEM((NSLOT, CHUNK, D), jnp.float32),
            pltpu.SemaphoreType.DMA((NSLOT,)),
            pltpu.SemaphoreType.DMA((NSLOT,)),
            pltpu.SemaphoreType.DMA((NSLOT,)),
        ],
    )
    def k(x_hbm, src_hbm, out_hbm, i0, i1, i2, i3, rows, sem_i, sem_g, sem_o):
        del i1, i2, i3, sem_i, sem_o
        c = lax.axis_index("c")
        s = lax.axis_index("s")
        wid = s * NC + c
        for j in range(NJ):
            base = (wid + NW * j) * CHUNK
            pltpu.sync_copy(src_hbm.at[pl.ds(base, CHUNK)], i0)
            pltpu.async_copy(x_hbm.at[i0], rows.at[0], sem_g.at[0]).wait()
            pltpu.sync_copy(rows.at[0], out_hbm.at[pl.ds(base, CHUNK)])

    return k(x_pad, src)


def _sc_scatter(msg, dst, zrows):
    mesh = plsc.VectorSubcoreMesh(core_axis_name="c", subcore_axis_name="s")

    @functools.partial(
        pl.kernel, mesh=mesh,
        out_type=jax.ShapeDtypeStruct((NC, N_PAD, D), jnp.float32),
        scratch_types=[
            pltpu.VMEM_SHARED((N_PAD, D), jnp.float32),
            *_IDX_SCRATCH2,
            pltpu.VMEM((SSLOT, CHUNK, D), jnp.float32),
            pltpu.SemaphoreType.DMA((SSLOT,)),
            pltpu.SemaphoreType.DMA((SSLOT,)),
            pltpu.SemaphoreType.DMA((SSLOT,)),
        ],
    )
    def k(msg_hbm, dst_hbm, zero_hbm, out_hbm, accum, i0, i1, rows,
          sem_i, sem_m, sem_a):
        idx = [i0, i1]
        c = lax.axis_index("c")
        s = lax.axis_index("s")
        wid = s * NC + c
        start = wid * NJ
        pltpu.sync_copy(zero_hbm, accum.at[pl.ds(s * ZROWS, ZROWS)])
        plsc.subcore_barrier()
        ih = [None] * NJ
        mh = [None] * NJ
        ah = [None] * NJ
        for t in range(NJ + 1):
            if t < NJ:
                b = t % SSLOT
                if t >= SSLOT:
                    ah[t - SSLOT].wait()
                ih[t] = pltpu.async_copy(
                    dst_hbm.at[pl.ds((start + t) * CHUNK, CHUNK)],
                    idx[b], sem_i.at[b])
                mh[t] = pltpu.async_copy(
                    msg_hbm.at[pl.ds((start + t) * CHUNK, CHUNK)],
                    rows.at[b], sem_m.at[b])
            ta = t - 1
            if 0 <= ta < NJ:
                ba = ta % SSLOT
                ih[ta].wait()
                mh[ta].wait()
                ah[ta] = pltpu.async_copy(
                    rows.at[ba], accum.at[idx[ba]], sem_a.at[ba], add=True)
        for ta in range(NJ - SSLOT, NJ):
            ah[ta].wait()
        plsc.subcore_barrier()
        pltpu.sync_copy(accum.at[pl.ds(s * ZROWS, ZROWS)],
                        out_hbm.at[c, pl.ds(s * ZROWS, ZROWS)])

    return k(msg, dst, zrows)


def _msg_body(xs_ref, r_ref, norm_ref, basis_ref, comp_ref, out_ref):
    xt = xs_ref[:].astype(jnp.bfloat16)
    rt = r_ref[0]
    nt = norm_ref[0]
    onehot_t = (rt == lax.broadcasted_iota(jnp.int32, (R, TE), 0))
    onehot_t = (onehot_t.astype(jnp.float32) * nt).astype(jnp.bfloat16)
    cw = lax.dot_general(onehot_t, comp_ref[:],
                         (((0,), (0,)), ((), ())),
                         preferred_element_type=jnp.float32).astype(jnp.bfloat16)
    xnw = jnp.concatenate([xt] * B, axis=1) * cw
    out_ref[:] = jnp.dot(xnw, basis_ref[:],
                         preferred_element_type=jnp.float32)


def _tc_msg(xs, r3, norm3, basis_flat, comp_wide):
    return pl.pallas_call(
        _msg_body,
        grid=(EH_P // TE,),
        in_specs=[
            pl.BlockSpec((TE, D), lambda i: (i, 0)),
            pl.BlockSpec((1, 1, TE), lambda i: (i, 0, 0)),
            pl.BlockSpec((1, 1, TE), lambda i: (i, 0, 0)),
            pl.BlockSpec((B * D, D), lambda i: (0, 0)),
            pl.BlockSpec((R, B * D), lambda i: (0, 0)),
        ],
        out_specs=pl.BlockSpec((TE, D), lambda i: (i, 0)),
        out_shape=jax.ShapeDtypeStruct((EH_P, D), jnp.float32),
    )(xs, r3, norm3, basis_flat, comp_wide)


def _final_body(act, pa_ref, pb_ref, x_ref, loop_ref, bias_ref, out_ref):
    pre = (pa_ref[0] + pa_ref[1] + pb_ref[0] + pb_ref[1]
           + jnp.dot(x_ref[:], loop_ref[:], preferred_element_type=jnp.float32)
           + bias_ref[:])
    out_ref[:] = act(pre)


def _tc_final(parts_a, parts_b, x_pad, loop_w, bias2d, act):
    return pl.pallas_call(
        functools.partial(_final_body, act),
        grid=(N_PAD // TN,),
        in_specs=[
            pl.BlockSpec((NC, TN, D), lambda i: (0, i, 0)),
            pl.BlockSpec((NC, TN, D), lambda i: (0, i, 0)),
            pl.BlockSpec((TN, D), lambda i: (i, 0)),
            pl.BlockSpec((D, D), lambda i: (0, 0)),
            pl.BlockSpec((1, D), lambda i: (0, 0)),
        ],
        out_specs=pl.BlockSpec((TN, D), lambda i: (i, 0)),
        out_shape=jax.ShapeDtypeStruct((N_PAD, D), jnp.float32),
    )(parts_a, parts_b, x_pad, loop_w, bias2d)


def _layer(x_pad, halves, zrows, basis, comp, loop_w, bias, act):
    basis_flat = basis.reshape(B * D, D).astype(jnp.bfloat16)
    comp_wide = jnp.repeat(comp, D, axis=1).astype(jnp.bfloat16)
    msgs = [None, None]
    for i, (src_h, dst_h, r3_h, n3_h) in enumerate(halves):
        xs = _sc_gather(x_pad, src_h)
        msgs[i] = _tc_msg(xs, r3_h, n3_h, basis_flat, comp_wide)
    parts = [_sc_scatter(msgs[i], h[1], zrows) for i, h in enumerate(halves)]
    return _tc_final(parts[0], parts[1], x_pad, loop_w,
                     bias.reshape(1, D), act)


def kernel(h, edge_index, r, norm, emb, basis1, comp1, loop1, bias1,
           basis2, comp2, loop2, bias2):
    src = edge_index[0]
    dst = edge_index[1]
    x = jnp.take(emb, h, axis=0)
    x_pad = jnp.pad(x, ((0, N_PAD - N), (0, 0)))
    pad_e = EH_P - EH
    halves = []
    for i in range(2):
        sl = slice(i * EH, (i + 1) * EH)
        src_p = jnp.pad(src[sl], (0, pad_e))
        dst_p = jnp.pad(dst[sl], (0, pad_e))
        r_p = jnp.pad(r[sl], (0, pad_e)).reshape(EH_P // TE, 1, TE)
        n_p = jnp.pad(norm[sl, 0], (0, pad_e)).reshape(EH_P // TE, 1, TE)
        halves.append((src_p, dst_p, r_p, n_p))
    zrows = jnp.zeros((ZROWS, D), jnp.float32)
    x_pad = _layer(x_pad, halves, zrows, basis1, comp1, loop1, bias1,
                   jax.nn.relu)
    x_pad = _layer(x_pad, halves, zrows, basis2, comp2, loop2, bias2,
                   jax.nn.sigmoid)
    return x_pad[:N]


# speedup vs baseline: 1.0873x; 1.0058x over previous
import functools

import jax
import jax.numpy as jnp
from jax import lax
from jax.experimental import pallas as pl
from jax.experimental.pallas import tpu as pltpu
from jax.experimental.pallas import tpu_sc as plsc

N, E, D, R, B = 10000, 160000, 128, 64, 8
N_PAD = 10240
EH = E // 2
EH_P = 81920
TE = 2560
TN = 640
CHUNK = 128
NC, NS = 2, 16
NW = NC * NS
NJ = EH_P // CHUNK // NW
NSLOT = 4
LOOK = 2
ZROWS = N_PAD // NS


SSLOT = 2
_IDX_SCRATCH = [pltpu.VMEM((CHUNK,), jnp.int32) for _ in range(NSLOT)]
_IDX_SCRATCH2 = [pltpu.VMEM((CHUNK,), jnp.int32) for _ in range(SSLOT)]


def _sc_gather(x_pad, src):
    mesh = plsc.VectorSubcoreMesh(core_axis_name="c", subcore_axis_name="s")

    @functools.partial(
        pl.kernel, mesh=mesh,
        out_type=jax.ShapeDtypeStruct((EH_P, D), jnp.float32),
        scratch_types=[
            pltpu.VMEM((CHUNK,), jnp.int32),
            pltpu.VMEM((CHUNK, D), jnp.float32),
            pltpu.SemaphoreType.DMA,
        ],
    )
    def k(x_hbm, src_hbm, out_hbm, idx_v, rows_v, sem):
        c = lax.axis_index("c")
        s = lax.axis_index("s")
        wid = s * NC + c

        def body(j, carry):
            base = (wid + NW * j) * CHUNK
            pltpu.sync_copy(src_hbm.at[pl.ds(base, CHUNK)], idx_v)
            pltpu.async_copy(x_hbm.at[idx_v], rows_v, sem).wait()
            pltpu.sync_copy(rows_v, out_hbm.at[pl.ds(base, CHUNK)])
            return carry

        lax.fori_loop(0, NJ, body, 0)

    return k(x_pad, src)


def _sc_scatter(msg, dst, zrows):
    mesh = plsc.VectorSubcoreMesh(core_axis_name="c", subcore_axis_name="s")

    @functools.partial(
        pl.kernel, mesh=mesh,
        out_type=jax.ShapeDtypeStruct((NC, N_PAD, D), jnp.float32),
        scratch_types=[
            pltpu.VMEM_SHARED((N_PAD, D), jnp.float32),
            *_IDX_SCRATCH2,
            pltpu.VMEM((SSLOT, CHUNK, D), jnp.float32),
            pltpu.SemaphoreType.DMA((SSLOT,)),
            pltpu.SemaphoreType.DMA((SSLOT,)),
            pltpu.SemaphoreType.DMA((SSLOT,)),
        ],
    )
    def k(msg_hbm, dst_hbm, zero_hbm, out_hbm, accum, i0, i1, rows,
          sem_i, sem_m, sem_a):
        idx = [i0, i1]
        c = lax.axis_index("c")
        s = lax.axis_index("s")
        wid = s * NC + c
        start = wid * NJ
        pltpu.sync_copy(zero_hbm, accum.at[pl.ds(s * ZROWS, ZROWS)])
        plsc.subcore_barrier()
        ih = [None] * NJ
        mh = [None] * NJ
        ah = [None] * NJ
        for t in range(NJ + 1):
            if t < NJ:
                b = t % SSLOT
                if t >= SSLOT:
                    ah[t - SSLOT].wait()
                ih[t] = pltpu.async_copy(
                    dst_hbm.at[pl.ds((start + t) * CHUNK, CHUNK)],
                    idx[b], sem_i.at[b])
                mh[t] = pltpu.async_copy(
                    msg_hbm.at[pl.ds((start + t) * CHUNK, CHUNK)],
                    rows.at[b], sem_m.at[b])
            ta = t - 1
            if 0 <= ta < NJ:
                ba = ta % SSLOT
                ih[ta].wait()
                mh[ta].wait()
                ah[ta] = pltpu.async_copy(
                    rows.at[ba], accum.at[idx[ba]], sem_a.at[ba], add=True)
        for ta in range(NJ - SSLOT, NJ):
            ah[ta].wait()
        plsc.subcore_barrier()
        pltpu.sync_copy(accum.at[pl.ds(s * ZROWS, ZROWS)],
                        out_hbm.at[c, pl.ds(s * ZROWS, ZROWS)])

    return k(msg, dst, zrows)


def _msg_body(xs_ref, r_ref, norm_ref, basis_ref, comp_ref, out_ref):
    xt = xs_ref[:].astype(jnp.bfloat16)
    rt = r_ref[0]
    nt = norm_ref[0]
    onehot_t = (rt == lax.broadcasted_iota(jnp.int32, (R, TE), 0))
    onehot_t = (onehot_t.astype(jnp.float32) * nt).astype(jnp.bfloat16)
    cw = lax.dot_general(onehot_t, comp_ref[:],
                         (((0,), (0,)), ((), ())),
                         preferred_element_type=jnp.float32).astype(jnp.bfloat16)
    xnw = jnp.concatenate([xt] * B, axis=1) * cw
    out_ref[:] = jnp.dot(xnw, basis_ref[:],
                         preferred_element_type=jnp.float32)


def _tc_msg(xs, r3, norm3, basis_flat, comp_wide):
    return pl.pallas_call(
        _msg_body,
        grid=(EH_P // TE,),
        in_specs=[
            pl.BlockSpec((TE, D), lambda i: (i, 0)),
            pl.BlockSpec((1, 1, TE), lambda i: (i, 0, 0)),
            pl.BlockSpec((1, 1, TE), lambda i: (i, 0, 0)),
            pl.BlockSpec((B * D, D), lambda i: (0, 0)),
            pl.BlockSpec((R, B * D), lambda i: (0, 0)),
        ],
        out_specs=pl.BlockSpec((TE, D), lambda i: (i, 0)),
        out_shape=jax.ShapeDtypeStruct((EH_P, D), jnp.float32),
    )(xs, r3, norm3, basis_flat, comp_wide)


def _final_body(act, pa_ref, pb_ref, x_ref, loop_ref, bias_ref, out_ref):
    pre = (pa_ref[0] + pa_ref[1] + pb_ref[0] + pb_ref[1]
           + jnp.dot(x_ref[:], loop_ref[:], preferred_element_type=jnp.float32)
           + bias_ref[:])
    out_ref[:] = act(pre)


def _tc_final(parts_a, parts_b, x_pad, loop_w, bias2d, act):
    return pl.pallas_call(
        functools.partial(_final_body, act),
        grid=(N_PAD // TN,),
        in_specs=[
            pl.BlockSpec((NC, TN, D), lambda i: (0, i, 0)),
            pl.BlockSpec((NC, TN, D), lambda i: (0, i, 0)),
            pl.BlockSpec((TN, D), lambda i: (i, 0)),
            pl.BlockSpec((D, D), lambda i: (0, 0)),
            pl.BlockSpec((1, D), lambda i: (0, 0)),
        ],
        out_specs=pl.BlockSpec((TN, D), lambda i: (i, 0)),
        out_shape=jax.ShapeDtypeStruct((N_PAD, D), jnp.float32),
    )(parts_a, parts_b, x_pad, loop_w, bias2d)


def _layer(x_pad, halves, zrows, basis, comp, loop_w, bias, act):
    basis_flat = basis.reshape(B * D, D).astype(jnp.bfloat16)
    comp_wide = jnp.repeat(comp, D, axis=1).astype(jnp.bfloat16)
    msgs = [None, None]
    for i, (src_h, dst_h, r3_h, n3_h) in enumerate(halves):
        xs = _sc_gather(x_pad, src_h)
        msgs[i] = _tc_msg(xs, r3_h, n3_h, basis_flat, comp_wide)
    parts = [_sc_scatter(msgs[i], h[1], zrows) for i, h in enumerate(halves)]
    return _tc_final(parts[0], parts[1], x_pad, loop_w,
                     bias.reshape(1, D), act)


def kernel(h, edge_index, r, norm, emb, basis1, comp1, loop1, bias1,
           basis2, comp2, loop2, bias2):
    src = edge_index[0]
    dst = edge_index[1]
    x = jnp.take(emb, h, axis=0)
    x_pad = jnp.pad(x, ((0, N_PAD - N), (0, 0)))
    pad_e = EH_P - EH
    halves = []
    for i in range(2):
        sl = slice(i * EH, (i + 1) * EH)
        src_p = jnp.pad(src[sl], (0, pad_e))
        dst_p = jnp.pad(dst[sl], (0, pad_e))
        r_p = jnp.pad(r[sl], (0, pad_e)).reshape(EH_P // TE, 1, TE)
        n_p = jnp.pad(norm[sl, 0], (0, pad_e)).reshape(EH_P // TE, 1, TE)
        halves.append((src_p, dst_p, r_p, n_p))
    zrows = jnp.zeros((ZROWS, D), jnp.float32)
    x_pad = _layer(x_pad, halves, zrows, basis1, comp1, loop1, bias1,
                   jax.nn.relu)
    x_pad = _layer(x_pad, halves, zrows, basis2, comp2, loop2, bias2,
                   jax.nn.sigmoid)
    return x_pad[:N]


# speedup vs baseline: 1.0876x; 1.0003x over previous
import functools

import jax
import jax.numpy as jnp
from jax import lax
from jax.experimental import pallas as pl
from jax.experimental.pallas import tpu as pltpu
from jax.experimental.pallas import tpu_sc as plsc

N, E, D, R, B = 10000, 160000, 128, 64, 8
N_PAD = 10240
EH = E // 2
EH_P = 81920
TE = 2560
TN = 640
CHUNK = 128
NC, NS = 2, 16
NW = NC * NS
NJ = EH_P // CHUNK // NW
NSLOT = 4
LOOK = 2
ZROWS = N_PAD // NS


SSLOT = 2
_IDX_SCRATCH = [pltpu.VMEM((CHUNK,), jnp.int32) for _ in range(NSLOT)]
_IDX_SCRATCH2 = [pltpu.VMEM((CHUNK,), jnp.int32) for _ in range(SSLOT)]


def _sc_gather(x_pad, src):
    mesh = plsc.VectorSubcoreMesh(core_axis_name="c", subcore_axis_name="s")

    @functools.partial(
        pl.kernel, mesh=mesh,
        out_type=jax.ShapeDtypeStruct((EH_P, D), jnp.float32),
        scratch_types=[
            pltpu.VMEM((CHUNK,), jnp.int32),
            pltpu.VMEM((CHUNK, D), jnp.float32),
            pltpu.SemaphoreType.DMA,
        ],
    )
    def k(x_hbm, src_hbm, out_hbm, idx_v, rows_v, sem):
        c = lax.axis_index("c")
        s = lax.axis_index("s")
        wid = s * NC + c

        def body(j, carry):
            base = (wid + NW * j) * CHUNK
            pltpu.sync_copy(src_hbm.at[pl.ds(base, CHUNK)], idx_v)
            pltpu.async_copy(x_hbm.at[idx_v], rows_v, sem).wait()
            pltpu.sync_copy(rows_v, out_hbm.at[pl.ds(base, CHUNK)])
            return carry

        nj = jnp.where(wid >= 0, NJ, 0)
        lax.fori_loop(0, nj, body, 0)

    return k(x_pad, src)


def _sc_scatter(msg, dst, zrows):
    mesh = plsc.VectorSubcoreMesh(core_axis_name="c", subcore_axis_name="s")

    @functools.partial(
        pl.kernel, mesh=mesh,
        out_type=jax.ShapeDtypeStruct((NC, N_PAD, D), jnp.float32),
        scratch_types=[
            pltpu.VMEM_SHARED((N_PAD, D), jnp.float32),
            *_IDX_SCRATCH2,
            pltpu.VMEM((SSLOT, CHUNK, D), jnp.float32),
            pltpu.SemaphoreType.DMA((SSLOT,)),
            pltpu.SemaphoreType.DMA((SSLOT,)),
            pltpu.SemaphoreType.DMA((SSLOT,)),
        ],
    )
    def k(msg_hbm, dst_hbm, zero_hbm, out_hbm, accum, i0, i1, rows,
          sem_i, sem_m, sem_a):
        idx = [i0, i1]
        c = lax.axis_index("c")
        s = lax.axis_index("s")
        wid = s * NC + c
        start = wid * NJ
        pltpu.sync_copy(zero_hbm, accum.at[pl.ds(s * ZROWS, ZROWS)])
        plsc.subcore_barrier()
        ih = [None] * NJ
        mh = [None] * NJ
        ah = [None] * NJ
        for t in range(NJ + 1):
            if t < NJ:
                b = t % SSLOT
                if t >= SSLOT:
                    ah[t - SSLOT].wait()
                ih[t] = pltpu.async_copy(
                    dst_hbm.at[pl.ds((start + t) * CHUNK, CHUNK)],
                    idx[b], sem_i.at[b])
                mh[t] = pltpu.async_copy(
                    msg_hbm.at[pl.ds((start + t) * CHUNK, CHUNK)],
                    rows.at[b], sem_m.at[b])
            ta = t - 1
            if 0 <= ta < NJ:
                ba = ta % SSLOT
                ih[ta].wait()
                mh[ta].wait()
                ah[ta] = pltpu.async_copy(
                    rows.at[ba], accum.at[idx[ba]], sem_a.at[ba], add=True)
        for ta in range(NJ - SSLOT, NJ):
            ah[ta].wait()
        plsc.subcore_barrier()
        pltpu.sync_copy(accum.at[pl.ds(s * ZROWS, ZROWS)],
                        out_hbm.at[c, pl.ds(s * ZROWS, ZROWS)])

    return k(msg, dst, zrows)


def _msg_body(xs_ref, r_ref, norm_ref, basis_ref, comp_ref, out_ref):
    xt = xs_ref[:].astype(jnp.bfloat16)
    rt = r_ref[0]
    nt = norm_ref[0]
    onehot_t = (rt == lax.broadcasted_iota(jnp.int32, (R, TE), 0))
    onehot_t = (onehot_t.astype(jnp.float32) * nt).astype(jnp.bfloat16)
    cw = lax.dot_general(onehot_t, comp_ref[:],
                         (((0,), (0,)), ((), ())),
                         preferred_element_type=jnp.float32).astype(jnp.bfloat16)
    xnw = jnp.concatenate([xt] * B, axis=1) * cw
    out_ref[:] = jnp.dot(xnw, basis_ref[:],
                         preferred_element_type=jnp.float32)


def _tc_msg(xs, r3, norm3, basis_flat, comp_wide):
    return pl.pallas_call(
        _msg_body,
        grid=(EH_P // TE,),
        in_specs=[
            pl.BlockSpec((TE, D), lambda i: (i, 0)),
            pl.BlockSpec((1, 1, TE), lambda i: (i, 0, 0)),
            pl.BlockSpec((1, 1, TE), lambda i: (i, 0, 0)),
            pl.BlockSpec((B * D, D), lambda i: (0, 0)),
            pl.BlockSpec((R, B * D), lambda i: (0, 0)),
        ],
        out_specs=pl.BlockSpec((TE, D), lambda i: (i, 0)),
        out_shape=jax.ShapeDtypeStruct((EH_P, D), jnp.float32),
    )(xs, r3, norm3, basis_flat, comp_wide)


def _final_body(act, pa_ref, pb_ref, x_ref, loop_ref, bias_ref, out_ref):
    pre = (pa_ref[0] + pa_ref[1] + pb_ref[0] + pb_ref[1]
           + jnp.dot(x_ref[:], loop_ref[:], preferred_element_type=jnp.float32)
           + bias_ref[:])
    out_ref[:] = act(pre)


def _tc_final(parts_a, parts_b, x_pad, loop_w, bias2d, act):
    return pl.pallas_call(
        functools.partial(_final_body, act),
        grid=(N_PAD // TN,),
        in_specs=[
            pl.BlockSpec((NC, TN, D), lambda i: (0, i, 0)),
            pl.BlockSpec((NC, TN, D), lambda i: (0, i, 0)),
            pl.BlockSpec((TN, D), lambda i: (i, 0)),
            pl.BlockSpec((D, D), lambda i: (0, 0)),
            pl.BlockSpec((1, D), lambda i: (0, 0)),
        ],
        out_specs=pl.BlockSpec((TN, D), lambda i: (i, 0)),
        out_shape=jax.ShapeDtypeStruct((N_PAD, D), jnp.float32),
    )(parts_a, parts_b, x_pad, loop_w, bias2d)


def _layer(x_pad, halves, zrows, basis, comp, loop_w, bias, act):
    basis_flat = basis.reshape(B * D, D).astype(jnp.bfloat16)
    comp_wide = jnp.repeat(comp, D, axis=1).astype(jnp.bfloat16)
    msgs = [None, None]
    for i, (src_h, dst_h, r3_h, n3_h) in enumerate(halves):
        xs = _sc_gather(x_pad, src_h)
        msgs[i] = _tc_msg(xs, r3_h, n3_h, basis_flat, comp_wide)
    parts = [_sc_scatter(msgs[i], h[1], zrows) for i, h in enumerate(halves)]
    return _tc_final(parts[0], parts[1], x_pad, loop_w,
                     bias.reshape(1, D), act)


def kernel(h, edge_index, r, norm, emb, basis1, comp1, loop1, bias1,
           basis2, comp2, loop2, bias2):
    src = edge_index[0]
    dst = edge_index[1]
    x = jnp.take(emb, h, axis=0)
    x_pad = jnp.pad(x, ((0, N_PAD - N), (0, 0)))
    pad_e = EH_P - EH
    halves = []
    for i in range(2):
        sl = slice(i * EH, (i + 1) * EH)
        src_p = jnp.pad(src[sl], (0, pad_e))
        dst_p = jnp.pad(dst[sl], (0, pad_e))
        r_p = jnp.pad(r[sl], (0, pad_e)).reshape(EH_P // TE, 1, TE)
        n_p = jnp.pad(norm[sl, 0], (0, pad_e)).reshape(EH_P // TE, 1, TE)
        halves.append((src_p, dst_p, r_p, n_p))
    zrows = jnp.zeros((ZROWS, D), jnp.float32)
    x_pad = _layer(x_pad, halves, zrows, basis1, comp1, loop1, bias1,
                   jax.nn.relu)
    x_pad = _layer(x_pad, halves, zrows, basis2, comp2, loop2, bias2,
                   jax.nn.sigmoid)
    return x_pad[:N]


# speedup vs baseline: 1.4499x; 1.3331x over previous
import functools

import jax
import jax.numpy as jnp
from jax import lax
from jax.experimental import pallas as pl
from jax.experimental.pallas import tpu as pltpu
from jax.experimental.pallas import tpu_sc as plsc

N, E, D, R, B = 10000, 160000, 128, 64, 8
N_PAD = 10240
EH = E // 2
EH_P = 81920
TE = 2560
TN = 640
CHUNK = 128
NC, NS = 2, 16
NW = NC * NS
NJ = EH_P // CHUNK // NW
NCH_SC = 626
SNJ = NCH_SC // NW
SEXTRA = NCH_SC % NW
NSLOT = 4
LOOK = 2
ZROWS = N_PAD // NS


SSLOT = 2
_IDX_SCRATCH = [pltpu.VMEM((CHUNK,), jnp.int32) for _ in range(NSLOT)]
_IDX_SCRATCH2 = [pltpu.VMEM((CHUNK,), jnp.int32) for _ in range(SSLOT)]


def _sc_gather(x_pad, src):
    mesh = plsc.VectorSubcoreMesh(core_axis_name="c", subcore_axis_name="s")

    @functools.partial(
        pl.kernel, mesh=mesh,
        out_type=jax.ShapeDtypeStruct((EH_P, D), jnp.float32),
        scratch_types=[
            pltpu.VMEM((CHUNK,), jnp.int32),
            pltpu.VMEM((CHUNK, D), jnp.float32),
            pltpu.SemaphoreType.DMA,
        ],
    )
    def k(x_hbm, src_hbm, out_hbm, idx_v, rows_v, sem):
        c = lax.axis_index("c")
        s = lax.axis_index("s")
        wid = s * NC + c

        def body(j, carry):
            base = (wid + NW * j) * CHUNK
            pltpu.sync_copy(src_hbm.at[pl.ds(base, CHUNK)], idx_v)
            pltpu.async_copy(x_hbm.at[idx_v], rows_v, sem).wait()
            pltpu.sync_copy(rows_v, out_hbm.at[pl.ds(base, CHUNK)])
            return carry

        nj = jnp.where(wid < SEXTRA, SNJ + 1, SNJ)
        lax.fori_loop(0, nj, body, 0)

    return k(x_pad, src)


def _sc_scatter(msg, dst, zrows):
    mesh = plsc.VectorSubcoreMesh(core_axis_name="c", subcore_axis_name="s")

    @functools.partial(
        pl.kernel, mesh=mesh,
        out_type=jax.ShapeDtypeStruct((NC, N_PAD, D), jnp.float32),
        scratch_types=[
            pltpu.VMEM_SHARED((N_PAD, D), jnp.float32),
            *_IDX_SCRATCH2,
            pltpu.VMEM((SSLOT, CHUNK, D), jnp.float32),
            pltpu.SemaphoreType.DMA((SSLOT,)),
            pltpu.SemaphoreType.DMA((SSLOT,)),
            pltpu.SemaphoreType.DMA((SSLOT,)),
        ],
    )
    def k(msg_hbm, dst_hbm, zero_hbm, out_hbm, accum, i0, i1, rows,
          sem_i, sem_m, sem_a):
        idx = [i0, i1]
        c = lax.axis_index("c")
        s = lax.axis_index("s")
        wid = s * NC + c
        start = (wid * SNJ + jnp.minimum(wid, SEXTRA)) * CHUNK
        pltpu.sync_copy(zero_hbm, accum.at[pl.ds(s * ZROWS, ZROWS)])
        plsc.subcore_barrier()
        ih = [None] * SNJ
        mh = [None] * SNJ
        ah = [None] * SNJ
        for t in range(SNJ + 1):
            if t < SNJ:
                b = t % SSLOT
                if t >= SSLOT:
                    ah[t - SSLOT].wait()
                ih[t] = pltpu.async_copy(
                    dst_hbm.at[pl.ds(start + t * CHUNK, CHUNK)],
                    idx[b], sem_i.at[b])
                mh[t] = pltpu.async_copy(
                    msg_hbm.at[pl.ds(start + t * CHUNK, CHUNK)],
                    rows.at[b], sem_m.at[b])
            ta = t - 1
            if 0 <= ta < SNJ:
                ba = ta % SSLOT
                ih[ta].wait()
                mh[ta].wait()
                ah[ta] = pltpu.async_copy(
                    rows.at[ba], accum.at[idx[ba]], sem_a.at[ba], add=True)
        for ta in range(SNJ - SSLOT, SNJ):
            ah[ta].wait()

        @pl.when(wid < SEXTRA)
        def _tail():
            base = start + SNJ * CHUNK
            pltpu.sync_copy(dst_hbm.at[pl.ds(base, CHUNK)], idx[0])
            pltpu.sync_copy(msg_hbm.at[pl.ds(base, CHUNK)], rows.at[0])
            pltpu.sync_copy(rows.at[0], accum.at[idx[0]], add=True)
        plsc.subcore_barrier()
        pltpu.sync_copy(accum.at[pl.ds(s * ZROWS, ZROWS)],
                        out_hbm.at[c, pl.ds(s * ZROWS, ZROWS)])

    return k(msg, dst, zrows)


def _msg_body(xs_ref, r_ref, norm_ref, basis_ref, comp_ref, out_ref):
    xt = xs_ref[:].astype(jnp.bfloat16)
    rt = r_ref[0]
    nt = norm_ref[0]
    onehot_t = (rt == lax.broadcasted_iota(jnp.int32, (R, TE), 0))
    onehot_t = (onehot_t.astype(jnp.float32) * nt).astype(jnp.bfloat16)
    cw = lax.dot_general(onehot_t, comp_ref[:],
                         (((0,), (0,)), ((), ())),
                         preferred_element_type=jnp.float32).astype(jnp.bfloat16)
    xnw = jnp.concatenate([xt] * B, axis=1) * cw
    out_ref[:] = jnp.dot(xnw, basis_ref[:],
                         preferred_element_type=jnp.float32)


def _tc_msg(xs, r3, norm3, basis_flat, comp_wide):
    return pl.pallas_call(
        _msg_body,
        grid=(EH_P // TE,),
        in_specs=[
            pl.BlockSpec((TE, D), lambda i: (i, 0)),
            pl.BlockSpec((1, 1, TE), lambda i: (i, 0, 0)),
            pl.BlockSpec((1, 1, TE), lambda i: (i, 0, 0)),
            pl.BlockSpec((B * D, D), lambda i: (0, 0)),
            pl.BlockSpec((R, B * D), lambda i: (0, 0)),
        ],
        out_specs=pl.BlockSpec((TE, D), lambda i: (i, 0)),
        out_shape=jax.ShapeDtypeStruct((EH_P, D), jnp.float32),
    )(xs, r3, norm3, basis_flat, comp_wide)


def _final_body(act, pa_ref, pb_ref, x_ref, loop_ref, bias_ref, out_ref):
    pre = (pa_ref[0] + pa_ref[1] + pb_ref[0] + pb_ref[1]
           + jnp.dot(x_ref[:], loop_ref[:], preferred_element_type=jnp.float32)
           + bias_ref[:])
    out_ref[:] = act(pre)


def _tc_final(parts_a, parts_b, x_pad, loop_w, bias2d, act):
    return pl.pallas_call(
        functools.partial(_final_body, act),
        grid=(N_PAD // TN,),
        in_specs=[
            pl.BlockSpec((NC, TN, D), lambda i: (0, i, 0)),
            pl.BlockSpec((NC, TN, D), lambda i: (0, i, 0)),
            pl.BlockSpec((TN, D), lambda i: (i, 0)),
            pl.BlockSpec((D, D), lambda i: (0, 0)),
            pl.BlockSpec((1, D), lambda i: (0, 0)),
        ],
        out_specs=pl.BlockSpec((TN, D), lambda i: (i, 0)),
        out_shape=jax.ShapeDtypeStruct((N_PAD, D), jnp.float32),
    )(parts_a, parts_b, x_pad, loop_w, bias2d)


def _layer(x_pad, halves, zrows, basis, comp, loop_w, bias, act):
    basis_flat = basis.reshape(B * D, D).astype(jnp.bfloat16)
    comp_wide = jnp.repeat(comp, D, axis=1).astype(jnp.bfloat16)
    msgs = [None, None]
    for i, (src_h, dst_h, r3_h, n3_h) in enumerate(halves):
        xs = _sc_gather(x_pad, src_h)
        msgs[i] = _tc_msg(xs, r3_h, n3_h, basis_flat, comp_wide)
    parts = [_sc_scatter(msgs[i], h[1], zrows) for i, h in enumerate(halves)]
    return _tc_final(parts[0], parts[1], x_pad, loop_w,
                     bias.reshape(1, D), act)


def kernel(h, edge_index, r, norm, emb, basis1, comp1, loop1, bias1,
           basis2, comp2, loop2, bias2):
    src = edge_index[0]
    dst = edge_index[1]
    x = jnp.take(emb, h, axis=0)
    x_pad = jnp.pad(x, ((0, N_PAD - N), (0, 0)))
    pad_e = EH_P - EH
    halves = []
    for i in range(2):
        sl = slice(i * EH, (i + 1) * EH)
        src_p = jnp.pad(src[sl], (0, pad_e))
        dst_p = jnp.pad(dst[sl], (0, pad_e))
        r_p = jnp.pad(r[sl], (0, pad_e)).reshape(EH_P // TE, 1, TE)
        n_p = jnp.pad(norm[sl, 0], (0, pad_e)).reshape(EH_P // TE, 1, TE)
        halves.append((src_p, dst_p, r_p, n_p))
    zrows = jnp.zeros((ZROWS, D), jnp.float32)
    x_pad = _layer(x_pad, halves, zrows, basis1, comp1, loop1, bias1,
                   jax.nn.relu)
    x_pad = _layer(x_pad, halves, zrows, basis2, comp2, loop2, bias2,
                   jax.nn.sigmoid)
    return x_pad[:N]
